# Initial kernel scaffold; baseline (speedup 1.0000x reference)
import jax
import jax.numpy as jnp
from jax.experimental import pallas as pl


def kernel(one_hot, features, gemme_features, a_res, We1, be1, We2, be2, We3, be3, Wg1, bg1, Wg2, bg2, Wg3, bg3, Wg4, bg4, Wg5, bg5, Wf1, bf1, Wf2, bf2, Wf3, bf3):
    raise NotImplementedError("write your pallas kernel here")


# speedup vs baseline: 9.3106x; 9.3106x over previous
import functools

import jax
import jax.numpy as jnp
from jax import lax
from jax.experimental import pallas as pl
from jax.experimental.pallas import tpu as pltpu
from jax.experimental.pallas import tpu_sc as plsc

_N = 10000
_E = 320000
_NC = 2
_NS = 16
_NW = _NC * _NS
_CHUNK = 128
_CPT = -(-(_E // _NW) // _CHUNK)
_EPT = _CPT * _CHUNK
_EPAD = _EPT * _NW
_NPAD = 10112
_ZR = _NPAD // _NS
_R = 1000
_G = _N // _R


def _elu(x):
    return jnp.where(x > 0, x, jnp.exp(jnp.minimum(x, 0.0)) - 1.0)


def _dot(a, b):
    return jnp.dot(a, b, preferred_element_type=jnp.float32)


@functools.lru_cache(maxsize=None)
def _make_seg_sum(dp):
    mesh = plsc.VectorSubcoreMesh(core_axis_name="c", subcore_axis_name="s",
                                  num_cores=_NC, num_subcores=_NS)

    @functools.partial(
        pl.kernel,
        out_type=jax.ShapeDtypeStruct((_NC, _NPAD, dp), jnp.float32),
        mesh=mesh,
        scratch_types=[
            pltpu.VMEM((_CPT, _CHUNK), jnp.int32),
            pltpu.VMEM((_CPT, _CHUNK), jnp.int32),
            pltpu.VMEM((_CHUNK, dp), jnp.float32),
            pltpu.VMEM_SHARED((_NPAD, dp), jnp.float32),
            pltpu.SemaphoreType.DMA,
        ],
        compiler_params=pltpu.CompilerParams(use_tc_tiling_on_sc=False),
    )
    def seg_sum(y, srcp, dstp, zeros, out, sidx, didx, rows, acc, sem):
        c = lax.axis_index("c")
        s = lax.axis_index("s")
        wid = s * _NC + c
        pltpu.sync_copy(zeros.at[pl.ds(s * _ZR, _ZR)], acc.at[pl.ds(s * _ZR, _ZR)])
        pltpu.sync_copy(srcp.at[wid], sidx)
        pltpu.sync_copy(dstp.at[wid], didx)
        plsc.subcore_barrier()

        def body(i, carry):
            pltpu.async_copy(y.at[sidx.at[i]], rows, sem).wait()
            pltpu.sync_copy(rows, acc.at[didx.at[i]], add=True)
            return carry

        lax.fori_loop(0, _CPT, body, 0)
        plsc.subcore_barrier()
        pltpu.sync_copy(acc.at[pl.ds(s * _ZR, _ZR)], out.at[c, pl.ds(s * _ZR, _ZR)])

    return seg_sum


def _full(shape):
    return pl.BlockSpec(shape, lambda i: (0,) * len(shape))


def _enc_body(oh, ft, w1a, w1b, b1, w2, b2, w3, b3, wg1, out):
    x = _elu(_dot(oh[...], w1a[...]) + _dot(ft[...], w1b[...]) + b1[...])
    x = _elu(_dot(x, w2[...]) + b2[...])
    x = _elu(_dot(x, w3[...]) + b3[...])
    out[...] = _dot(x, wg1[...])


def _encoder(oh, ft, w1a, w1b, b1, w2, b2, w3, b3, wg1):
    return pl.pallas_call(
        _enc_body,
        grid=(_G,),
        in_specs=[
            pl.BlockSpec((_R, 20), lambda i: (i, 0)),
            pl.BlockSpec((_R, 128), lambda i: (i, 0)),
        ] + [_full(a.shape) for a in (w1a, w1b, b1, w2, b2, w3, b3, wg1)],
        out_specs=pl.BlockSpec((_R, 64), lambda i: (i, 0)),
        out_shape=jax.ShapeDtypeStruct((_N, 64), jnp.float32),
    )(oh, ft, w1a, w1b, b1, w2, b2, w3, b3, wg1)


def _mid_body(p, b, w, out):
    x = _elu(p[0] + p[1] + b[...])
    out[...] = _dot(x, w[...])


def _mid(p, b, w):
    dk = p.shape[-1]
    dn = w.shape[-1]
    return pl.pallas_call(
        _mid_body,
        grid=(_G,),
        in_specs=[
            pl.BlockSpec((2, _R, dk), lambda i: (0, i, 0)),
            _full(b.shape), _full(w.shape),
        ],
        out_specs=pl.BlockSpec((_R, dn), lambda i: (i, 0)),
        out_shape=jax.ShapeDtypeStruct((_N, dn), jnp.float32),
    )(p, b, w)


def _fin_body(p, bg, w1, b1, w2, b2, w3, b3, out):
    x = _elu(p[0] + p[1] + bg[...])
    x = _elu(_dot(x, w1[...]) + b1[...])
    x = _elu(_dot(x, w2[...]) + b2[...])
    out[...] = jax.nn.sigmoid(_dot(x, w3[...]) + b3[...])


def _final(p, bg, w1, b1, w2, b2, w3, b3):
    return pl.pallas_call(
        _fin_body,
        grid=(_G,),
        in_specs=[pl.BlockSpec((2, _R, 16), lambda i: (0, i, 0))]
        + [_full(a.shape) for a in (bg, w1, b1, w2, b2, w3, b3)],
        out_specs=pl.BlockSpec((_R, 1), lambda i: (i, 0)),
        out_shape=jax.ShapeDtypeStruct((_N, 1), jnp.float32),
    )(p, bg, w1, b1, w2, b2, w3, b3)


def kernel(one_hot, features, gemme_features, a_res,
           We1, be1, We2, be2, We3, be3,
           Wg1, bg1, Wg2, bg2, Wg3, bg3, Wg4, bg4, Wg5, bg5,
           Wf1, bf1, Wf2, bf2, Wf3, bf3):
    pad = _EPAD - _E
    srcp = jnp.pad(a_res[0], (0, pad)).reshape(_NW, _CPT, _CHUNK)
    dstp = jnp.pad(a_res[1], (0, pad), constant_values=_N).reshape(_NW, _CPT, _CHUNK)

    row = lambda b: b.reshape(1, -1)
    Wg4p = jnp.pad(Wg4, ((0, 0), (0, 8)))
    bg4p = row(jnp.pad(bg4, (0, 8)))
    Wg5p = jnp.pad(Wg5, ((0, 8), (0, 12)))
    bg5p = row(jnp.pad(bg5, (0, 12)))
    Wf1p = jnp.pad(Wf1, ((0, 12), (0, 0)))

    z64 = jnp.zeros((_NPAD, 64), jnp.float32)
    z32 = jnp.zeros((_NPAD, 32), jnp.float32)
    z16 = jnp.zeros((_NPAD, 16), jnp.float32)

    y = _encoder(one_hot, features, We1[:20], We1[20:], row(be1),
                 We2, row(be2), We3, row(be3), Wg1)
    p = _make_seg_sum(64)(y, srcp, dstp, z64)
    y = _mid(p, row(bg1), Wg2)
    p = _make_seg_sum(32)(y, srcp, dstp, z32)
    y = _mid(p, row(bg2), Wg3)
    p = _make_seg_sum(16)(y, srcp, dstp, z16)
    y = _mid(p, row(bg3), Wg4p)
    p = _make_seg_sum(16)(y, srcp, dstp, z16)
    y = _mid(p, bg4p, Wg5p)
    p = _make_seg_sum(16)(y, srcp, dstp, z16)
    return _final(p, bg5p, Wf1p, row(bf1), Wf2, row(bf2), Wf3, row(bf3))


# speedup vs baseline: 9.3490x; 1.0041x over previous
import functools

import jax
import jax.numpy as jnp
from jax import lax
from jax.experimental import pallas as pl
from jax.experimental.pallas import tpu as pltpu
from jax.experimental.pallas import tpu_sc as plsc

_N = 10000
_E = 320000
_NC = 2
_NS = 16
_NW = _NC * _NS
_CHUNK = 128
_NB = 4
_CPT = 80
_EPT = _CPT * _CHUNK
_EPAD = _EPT * _NW
_NPAD = 10112
_ZR = _NPAD // _NS
_R = 1000
_G = _N // _R


def _elu(x):
    return jnp.where(x > 0, x, jnp.exp(jnp.minimum(x, 0.0)) - 1.0)


def _dot(a, b):
    return jnp.dot(a, b, preferred_element_type=jnp.float32)


@functools.lru_cache(maxsize=None)
def _make_seg_sum(dp):
    mesh = plsc.VectorSubcoreMesh(core_axis_name="c", subcore_axis_name="s",
                                  num_cores=_NC, num_subcores=_NS)

    @functools.partial(
        pl.kernel,
        out_type=jax.ShapeDtypeStruct((_NC, _NPAD, dp), jnp.float32),
        mesh=mesh,
        scratch_types=[
            pltpu.VMEM((_CPT, _CHUNK), jnp.int32),
            pltpu.VMEM((_CPT, _CHUNK), jnp.int32),
            [pltpu.VMEM((_CHUNK, dp), jnp.float32) for _ in range(_NB)],
            [pltpu.SemaphoreType.DMA for _ in range(_NB)],
            [pltpu.SemaphoreType.DMA for _ in range(_NB)],
            pltpu.VMEM_SHARED((_NPAD, dp), jnp.float32),
        ],
        compiler_params=pltpu.CompilerParams(use_tc_tiling_on_sc=False),
    )
    def seg_sum(y, srcp, dstp, zeros, out, sidx, didx, rows, gsem, ssem, acc):
        c = lax.axis_index("c")
        s = lax.axis_index("s")
        wid = s * _NC + c
        pltpu.sync_copy(zeros.at[pl.ds(s * _ZR, _ZR)], acc.at[pl.ds(s * _ZR, _ZR)])
        pltpu.sync_copy(srcp.at[wid], sidx)
        pltpu.sync_copy(dstp.at[wid], didx)
        plsc.subcore_barrier()

        for b in range(_NB):
            pltpu.async_copy(y.at[sidx.at[b]], rows[b], gsem[b])

        nj = _CPT // _NB

        def body(j, carry):
            base = j * _NB
            for b in range(_NB):
                i = base + b
                pltpu.make_async_copy(y.at[sidx.at[i]], rows[b], gsem[b]).wait()
                pltpu.async_copy(rows[b], acc.at[didx.at[i]], ssem[b], add=True)

            @pl.when(j + 1 < nj)
            def _():
                for b in range(_NB):
                    i = base + b
                    pltpu.make_async_copy(rows[b], acc.at[didx.at[i]], ssem[b]).wait()
                    pltpu.async_copy(y.at[sidx.at[i + _NB]], rows[b], gsem[b])

            return carry

        lax.fori_loop(0, nj, body, 0)
        for b in range(_NB):
            i = _CPT - _NB + b
            pltpu.make_async_copy(rows[b], acc.at[didx.at[i]], ssem[b]).wait()
        plsc.subcore_barrier()
        pltpu.sync_copy(acc.at[pl.ds(s * _ZR, _ZR)], out.at[c, pl.ds(s * _ZR, _ZR)])

    return seg_sum


def _full(shape):
    return pl.BlockSpec(shape, lambda i: (0,) * len(shape))


def _enc_body(oh, ft, w1a, w1b, b1, w2, b2, w3, b3, wg1, out):
    x = _elu(_dot(oh[...], w1a[...]) + _dot(ft[...], w1b[...]) + b1[...])
    x = _elu(_dot(x, w2[...]) + b2[...])
    x = _elu(_dot(x, w3[...]) + b3[...])
    out[...] = _dot(x, wg1[...])


def _encoder(oh, ft, w1a, w1b, b1, w2, b2, w3, b3, wg1):
    return pl.pallas_call(
        _enc_body,
        grid=(_G,),
        in_specs=[
            pl.BlockSpec((_R, 20), lambda i: (i, 0)),
            pl.BlockSpec((_R, 128), lambda i: (i, 0)),
        ] + [_full(a.shape) for a in (w1a, w1b, b1, w2, b2, w3, b3, wg1)],
        out_specs=pl.BlockSpec((_R, 64), lambda i: (i, 0)),
        out_shape=jax.ShapeDtypeStruct((_N, 64), jnp.float32),
    )(oh, ft, w1a, w1b, b1, w2, b2, w3, b3, wg1)


def _mid_body(p, b, w, out):
    x = _elu(p[0] + p[1] + b[...])
    out[...] = _dot(x, w[...])


def _mid(p, b, w):
    dk = p.shape[-1]
    dn = w.shape[-1]
    return pl.pallas_call(
        _mid_body,
        grid=(_G,),
        in_specs=[
            pl.BlockSpec((2, _R, dk), lambda i: (0, i, 0)),
            _full(b.shape), _full(w.shape),
        ],
        out_specs=pl.BlockSpec((_R, dn), lambda i: (i, 0)),
        out_shape=jax.ShapeDtypeStruct((_N, dn), jnp.float32),
    )(p, b, w)


def _fin_body(p, bg, w1, b1, w2, b2, w3, b3, out):
    x = _elu(p[0] + p[1] + bg[...])
    x = _elu(_dot(x, w1[...]) + b1[...])
    x = _elu(_dot(x, w2[...]) + b2[...])
    out[...] = jax.nn.sigmoid(_dot(x, w3[...]) + b3[...])


def _final(p, bg, w1, b1, w2, b2, w3, b3):
    return pl.pallas_call(
        _fin_body,
        grid=(_G,),
        in_specs=[pl.BlockSpec((2, _R, 16), lambda i: (0, i, 0))]
        + [_full(a.shape) for a in (bg, w1, b1, w2, b2, w3, b3)],
        out_specs=pl.BlockSpec((_R, 1), lambda i: (i, 0)),
        out_shape=jax.ShapeDtypeStruct((_N, 1), jnp.float32),
    )(p, bg, w1, b1, w2, b2, w3, b3)


def kernel(one_hot, features, gemme_features, a_res,
           We1, be1, We2, be2, We3, be3,
           Wg1, bg1, Wg2, bg2, Wg3, bg3, Wg4, bg4, Wg5, bg5,
           Wf1, bf1, Wf2, bf2, Wf3, bf3):
    pad = _EPAD - _E
    srcp = jnp.pad(a_res[0], (0, pad)).reshape(_NW, _CPT, _CHUNK)
    dstp = jnp.pad(a_res[1], (0, pad), constant_values=_N).reshape(_NW, _CPT, _CHUNK)

    row = lambda b: b.reshape(1, -1)
    Wg4p = jnp.pad(Wg4, ((0, 0), (0, 8)))
    bg4p = row(jnp.pad(bg4, (0, 8)))
    Wg5p = jnp.pad(Wg5, ((0, 8), (0, 12)))
    bg5p = row(jnp.pad(bg5, (0, 12)))
    Wf1p = jnp.pad(Wf1, ((0, 12), (0, 0)))

    z64 = jnp.zeros((_NPAD, 64), jnp.float32)
    z32 = jnp.zeros((_NPAD, 32), jnp.float32)
    z16 = jnp.zeros((_NPAD, 16), jnp.float32)

    y = _encoder(one_hot, features, We1[:20], We1[20:], row(be1),
                 We2, row(be2), We3, row(be3), Wg1)
    p = _make_seg_sum(64)(y, srcp, dstp, z64)
    y = _mid(p, row(bg1), Wg2)
    p = _make_seg_sum(32)(y, srcp, dstp, z32)
    y = _mid(p, row(bg2), Wg3)
    p = _make_seg_sum(16)(y, srcp, dstp, z16)
    y = _mid(p, row(bg3), Wg4p)
    p = _make_seg_sum(16)(y, srcp, dstp, z16)
    y = _mid(p, bg4p, Wg5p)
    p = _make_seg_sum(16)(y, srcp, dstp, z16)
    return _final(p, bg5p, Wf1p, row(bf1), Wf2, row(bf2), Wf3, row(bf3))


# speedup vs baseline: 10.1863x; 1.0896x over previous
import functools

import jax
import jax.numpy as jnp
from jax import lax
from jax.experimental import pallas as pl
from jax.experimental.pallas import tpu as pltpu
from jax.experimental.pallas import tpu_sc as plsc

_N = 10000
_E = 320000
_NC = 2
_NS = 16
_NW = _NC * _NS
_CHUNK = 128
_NB = 4
_CPT = 80
_EPT = _CPT * _CHUNK
_EPAD = _EPT * _NW
_NPAD = 10112
_ZR = _NPAD // _NS
_R = 1000
_G = _N // _R


def _elu(x):
    return jnp.where(x > 0, x, jnp.exp(jnp.minimum(x, 0.0)) - 1.0)


def _dot(a, b):
    return jnp.dot(a, b, preferred_element_type=jnp.float32)


@functools.lru_cache(maxsize=None)
def _make_seg_sum(dp):
    mesh = plsc.VectorSubcoreMesh(core_axis_name="c", subcore_axis_name="s",
                                  num_cores=_NC, num_subcores=_NS)

    @functools.partial(
        pl.kernel,
        out_type=jax.ShapeDtypeStruct((_NC, _NPAD, dp), jnp.float32),
        mesh=mesh,
        scratch_types=[
            pltpu.VMEM((_CPT, _CHUNK), jnp.int32),
            pltpu.VMEM((_CPT, _CHUNK), jnp.int32),
            [pltpu.VMEM((_CHUNK, dp), jnp.float32) for _ in range(_NB)],
            [pltpu.SemaphoreType.DMA for _ in range(_NB)],
            [pltpu.SemaphoreType.DMA for _ in range(_NB)],
            pltpu.VMEM_SHARED((_NPAD, dp), jnp.float32),
        ],
        compiler_params=pltpu.CompilerParams(use_tc_tiling_on_sc=False),
    )
    def seg_sum(y, srcp, dstp, zeros, out, sidx, didx, rows, gsem, ssem, acc):
        c = lax.axis_index("c")
        s = lax.axis_index("s")
        wid = s * _NC + c
        pltpu.sync_copy(zeros.at[pl.ds(s * _ZR, _ZR)], acc.at[pl.ds(s * _ZR, _ZR)])
        pltpu.sync_copy(srcp.at[wid], sidx)
        pltpu.sync_copy(dstp.at[wid], didx)
        plsc.subcore_barrier()

        for b in range(_NB):
            pltpu.async_copy(y.at[sidx.at[b]], rows[b], gsem[b])

        nj = _CPT // _NB

        def body(j, carry):
            base = j * _NB
            for b in range(_NB):
                i = base + b
                pltpu.make_async_copy(y.at[sidx.at[i]], rows[b], gsem[b]).wait()
                pltpu.async_copy(rows[b], acc.at[didx.at[i]], ssem[b], add=True)

            @pl.when(j + 1 < nj)
            def _():
                for b in range(_NB):
                    i = base + b
                    pltpu.make_async_copy(rows[b], acc.at[didx.at[i]], ssem[b]).wait()
                    pltpu.async_copy(y.at[sidx.at[i + _NB]], rows[b], gsem[b])

            return carry

        lax.fori_loop(0, nj, body, 0)
        for b in range(_NB):
            i = _CPT - _NB + b
            pltpu.make_async_copy(rows[b], acc.at[didx.at[i]], ssem[b]).wait()
        plsc.subcore_barrier()
        pltpu.sync_copy(acc.at[pl.ds(s * _ZR, _ZR)], out.at[c, pl.ds(s * _ZR, _ZR)])

    return seg_sum


def _full(shape):
    return pl.BlockSpec(shape, lambda i: (0,) * len(shape))


def _enc_body(oh, ft, w1a, w1b, b1, w2, b2, w3, b3, wg1, out):
    x = _elu(_dot(oh[...], w1a[...]) + _dot(ft[...], w1b[...]) + b1[...])
    x = _elu(_dot(x, w2[...]) + b2[...])
    x = _elu(_dot(x, w3[...]) + b3[...])
    out[...] = _dot(x, wg1[...])


def _encoder(oh, ft, w1a, w1b, b1, w2, b2, w3, b3, wg1):
    return pl.pallas_call(
        _enc_body,
        grid=(_G,),
        in_specs=[
            pl.BlockSpec((_R, 20), lambda i: (i, 0)),
            pl.BlockSpec((_R, 128), lambda i: (i, 0)),
        ] + [_full(a.shape) for a in (w1a, w1b, b1, w2, b2, w3, b3, wg1)],
        out_specs=pl.BlockSpec((_R, 64), lambda i: (i, 0)),
        out_shape=jax.ShapeDtypeStruct((_N, 64), jnp.float32),
    )(oh, ft, w1a, w1b, b1, w2, b2, w3, b3, wg1)


def _mid_body(p, b, w, out):
    x = _elu(p[0] + p[1] + b[...])
    out[...] = _dot(x, w[...])


def _mid(p, b, w):
    dk = p.shape[-1]
    dn = w.shape[-1]
    return pl.pallas_call(
        _mid_body,
        grid=(_G,),
        in_specs=[
            pl.BlockSpec((2, _R, dk), lambda i: (0, i, 0)),
            _full(b.shape), _full(w.shape),
        ],
        out_specs=pl.BlockSpec((_R, dn), lambda i: (i, 0)),
        out_shape=jax.ShapeDtypeStruct((_N, dn), jnp.float32),
    )(p, b, w)


def _fin_body(p, bg, w1, b1, w2, b2, w3, b3, out):
    x = _elu(p[0] + p[1] + bg[...])
    x = _elu(_dot(x, w1[...]) + b1[...])
    x = _elu(_dot(x, w2[...]) + b2[...])
    out[...] = jax.nn.sigmoid(_dot(x, w3[...]) + b3[...])


def _final(p, bg, w1, b1, w2, b2, w3, b3):
    return pl.pallas_call(
        _fin_body,
        grid=(_G,),
        in_specs=[pl.BlockSpec((2, _R, 16), lambda i: (0, i, 0))]
        + [_full(a.shape) for a in (bg, w1, b1, w2, b2, w3, b3)],
        out_specs=pl.BlockSpec((_R, 1), lambda i: (i, 0)),
        out_shape=jax.ShapeDtypeStruct((_N, 1), jnp.float32),
    )(p, bg, w1, b1, w2, b2, w3, b3)


def kernel(one_hot, features, gemme_features, a_res,
           We1, be1, We2, be2, We3, be3,
           Wg1, bg1, Wg2, bg2, Wg3, bg3, Wg4, bg4, Wg5, bg5,
           Wf1, bf1, Wf2, bf2, Wf3, bf3):
    ept_real = _E // _NW
    padc = _EPT - ept_real
    srcp = jnp.pad(a_res[0].reshape(_NW, ept_real),
                   ((0, 0), (0, padc))).reshape(_NW, _CPT, _CHUNK)
    trash = _N + (jnp.arange(padc, dtype=jnp.int32) % (_NPAD - _N))
    dstp = jnp.concatenate(
        [a_res[1].reshape(_NW, ept_real),
         jnp.broadcast_to(trash, (_NW, padc))], axis=1).reshape(_NW, _CPT, _CHUNK)

    row = lambda b: b.reshape(1, -1)
    Wg4p = jnp.pad(Wg4, ((0, 0), (0, 8)))
    bg4p = row(jnp.pad(bg4, (0, 8)))
    Wg5p = jnp.pad(Wg5, ((0, 8), (0, 12)))
    bg5p = row(jnp.pad(bg5, (0, 12)))
    Wf1p = jnp.pad(Wf1, ((0, 12), (0, 0)))

    z64 = jnp.zeros((_NPAD, 64), jnp.float32)
    z32 = jnp.zeros((_NPAD, 32), jnp.float32)
    z16 = jnp.zeros((_NPAD, 16), jnp.float32)

    y = _encoder(one_hot, features, We1[:20], We1[20:], row(be1),
                 We2, row(be2), We3, row(be3), Wg1)
    p = _make_seg_sum(64)(y, srcp, dstp, z64)
    y = _mid(p, row(bg1), Wg2)
    p = _make_seg_sum(32)(y, srcp, dstp, z32)
    y = _mid(p, row(bg2), Wg3)
    p = _make_seg_sum(16)(y, srcp, dstp, z16)
    y = _mid(p, row(bg3), Wg4p)
    p = _make_seg_sum(16)(y, srcp, dstp, z16)
    y = _mid(p, bg4p, Wg5p)
    p = _make_seg_sum(16)(y, srcp, dstp, z16)
    return _final(p, bg5p, Wf1p, row(bf1), Wf2, row(bf2), Wf3, row(bf3))


# speedup vs baseline: 18.9964x; 1.8649x over previous
import functools

import jax
import jax.numpy as jnp
from jax import lax
from jax.experimental import pallas as pl
from jax.experimental.pallas import tpu as pltpu
from jax.experimental.pallas import tpu_sc as plsc

_N = 10000
_E = 320000
_NC = 2
_NS = 16
_NW = _NC * _NS
_CHUNK = 125
_NB = 4
_CPT = 80
_EPT = _CPT * _CHUNK
_EPAD = _EPT * _NW
_NPAD = 10112
_ZR = _NPAD // _NS
_R = 1000
_G = _N // _R


def _elu(x):
    return jnp.where(x > 0, x, jnp.exp(jnp.minimum(x, 0.0)) - 1.0)


def _dot(a, b):
    return jnp.dot(a, b, preferred_element_type=jnp.float32)


@functools.lru_cache(maxsize=None)
def _make_seg_sum(dp):
    mesh = plsc.VectorSubcoreMesh(core_axis_name="c", subcore_axis_name="s",
                                  num_cores=_NC, num_subcores=_NS)

    @functools.partial(
        pl.kernel,
        out_type=jax.ShapeDtypeStruct((_NC, _NPAD, dp), jnp.float32),
        mesh=mesh,
        scratch_types=[
            pltpu.VMEM((_CPT, _CHUNK), jnp.int32),
            pltpu.VMEM((_CPT, _CHUNK), jnp.int32),
            [pltpu.VMEM((_CHUNK, dp), jnp.float32) for _ in range(_NB)],
            [pltpu.SemaphoreType.DMA for _ in range(_NB)],
            [pltpu.SemaphoreType.DMA for _ in range(_NB)],
            pltpu.VMEM_SHARED((_NPAD, dp), jnp.float32),
        ],
        compiler_params=pltpu.CompilerParams(use_tc_tiling_on_sc=False),
    )
    def seg_sum(y, srcp, dstp, zeros, out, sidx, didx, rows, gsem, ssem, acc):
        c = lax.axis_index("c")
        s = lax.axis_index("s")
        wid = s * _NC + c
        pltpu.sync_copy(zeros.at[pl.ds(s * _ZR, _ZR)], acc.at[pl.ds(s * _ZR, _ZR)])
        pltpu.sync_copy(srcp.at[wid], sidx)
        pltpu.sync_copy(dstp.at[wid], didx)
        plsc.subcore_barrier()

        for b in range(_NB):
            pltpu.async_copy(y.at[sidx.at[b]], rows[b], gsem[b])

        nj = _CPT // _NB

        def body(j, carry):
            base = j * _NB
            for b in range(_NB):
                i = base + b
                pltpu.make_async_copy(y.at[sidx.at[i]], rows[b], gsem[b]).wait()
                pltpu.async_copy(rows[b], acc.at[didx.at[i]], ssem[b], add=True)

            @pl.when(j + 1 < nj)
            def _():
                for b in range(_NB):
                    i = base + b
                    pltpu.make_async_copy(rows[b], acc.at[didx.at[i]], ssem[b]).wait()
                    pltpu.async_copy(y.at[sidx.at[i + _NB]], rows[b], gsem[b])

            return carry

        lax.fori_loop(0, nj, body, 0)
        for b in range(_NB):
            i = _CPT - _NB + b
            pltpu.make_async_copy(rows[b], acc.at[didx.at[i]], ssem[b]).wait()
        plsc.subcore_barrier()
        pltpu.sync_copy(acc.at[pl.ds(s * _ZR, _ZR)], out.at[c, pl.ds(s * _ZR, _ZR)])

    return seg_sum


def _full(shape):
    return pl.BlockSpec(shape, lambda i: (0,) * len(shape))


def _enc_body(oh, ft, w1a, w1b, b1, w2, b2, w3, b3, wg1, out):
    x = _elu(_dot(oh[...], w1a[...]) + _dot(ft[...], w1b[...]) + b1[...])
    x = _elu(_dot(x, w2[...]) + b2[...])
    x = _elu(_dot(x, w3[...]) + b3[...])
    out[...] = _dot(x, wg1[...])


def _encoder(oh, ft, w1a, w1b, b1, w2, b2, w3, b3, wg1):
    return pl.pallas_call(
        _enc_body,
        grid=(_G,),
        in_specs=[
            pl.BlockSpec((_R, 20), lambda i: (i, 0)),
            pl.BlockSpec((_R, 128), lambda i: (i, 0)),
        ] + [_full(a.shape) for a in (w1a, w1b, b1, w2, b2, w3, b3, wg1)],
        out_specs=pl.BlockSpec((_R, 64), lambda i: (i, 0)),
        out_shape=jax.ShapeDtypeStruct((_N, 64), jnp.float32),
    )(oh, ft, w1a, w1b, b1, w2, b2, w3, b3, wg1)


def _mid_body(p, b, w, out):
    x = _elu(p[0] + p[1] + b[...])
    out[...] = _dot(x, w[...])


def _mid(p, b, w):
    dk = p.shape[-1]
    dn = w.shape[-1]
    return pl.pallas_call(
        _mid_body,
        grid=(_G,),
        in_specs=[
            pl.BlockSpec((2, _R, dk), lambda i: (0, i, 0)),
            _full(b.shape), _full(w.shape),
        ],
        out_specs=pl.BlockSpec((_R, dn), lambda i: (i, 0)),
        out_shape=jax.ShapeDtypeStruct((_N, dn), jnp.float32),
    )(p, b, w)


def _fin_body(p, bg, w1, b1, w2, b2, w3, b3, out):
    x = _elu(p[0] + p[1] + bg[...])
    x = _elu(_dot(x, w1[...]) + b1[...])
    x = _elu(_dot(x, w2[...]) + b2[...])
    out[...] = jax.nn.sigmoid(_dot(x, w3[...]) + b3[...])


def _final(p, bg, w1, b1, w2, b2, w3, b3):
    return pl.pallas_call(
        _fin_body,
        grid=(_G,),
        in_specs=[pl.BlockSpec((2, _R, 16), lambda i: (0, i, 0))]
        + [_full(a.shape) for a in (bg, w1, b1, w2, b2, w3, b3)],
        out_specs=pl.BlockSpec((_R, 1), lambda i: (i, 0)),
        out_shape=jax.ShapeDtypeStruct((_N, 1), jnp.float32),
    )(p, bg, w1, b1, w2, b2, w3, b3)


def kernel(one_hot, features, gemme_features, a_res,
           We1, be1, We2, be2, We3, be3,
           Wg1, bg1, Wg2, bg2, Wg3, bg3, Wg4, bg4, Wg5, bg5,
           Wf1, bf1, Wf2, bf2, Wf3, bf3):
    srcp = a_res[0].reshape(_NW, _CPT, _CHUNK)
    dstp = a_res[1].reshape(_NW, _CPT, _CHUNK)

    row = lambda b: b.reshape(1, -1)
    Wg4p = jnp.pad(Wg4, ((0, 0), (0, 8)))
    bg4p = row(jnp.pad(bg4, (0, 8)))
    Wg5p = jnp.pad(Wg5, ((0, 8), (0, 12)))
    bg5p = row(jnp.pad(bg5, (0, 12)))
    Wf1p = jnp.pad(Wf1, ((0, 12), (0, 0)))

    z64 = jnp.zeros((_NPAD, 64), jnp.float32)
    z32 = jnp.zeros((_NPAD, 32), jnp.float32)
    z16 = jnp.zeros((_NPAD, 16), jnp.float32)

    y = _encoder(one_hot, features, We1[:20], We1[20:], row(be1),
                 We2, row(be2), We3, row(be3), Wg1)
    p = _make_seg_sum(64)(y, srcp, dstp, z64)
    y = _mid(p, row(bg1), Wg2)
    p = _make_seg_sum(32)(y, srcp, dstp, z32)
    y = _mid(p, row(bg2), Wg3)
    p = _make_seg_sum(16)(y, srcp, dstp, z16)
    y = _mid(p, row(bg3), Wg4p)
    p = _make_seg_sum(16)(y, srcp, dstp, z16)
    y = _mid(p, bg4p, Wg5p)
    p = _make_seg_sum(16)(y, srcp, dstp, z16)
    return _final(p, bg5p, Wf1p, row(bf1), Wf2, row(bf2), Wf3, row(bf3))


# speedup vs baseline: 20.4706x; 1.0776x over previous
import functools

import jax
import jax.numpy as jnp
from jax import lax
from jax.experimental import pallas as pl
from jax.experimental.pallas import tpu as pltpu
from jax.experimental.pallas import tpu_sc as plsc

_N = 10000
_E = 320000
_NC = 2
_NS = 16
_NW = _NC * _NS
_CHUNK = 125
_NB = 4
_CPT = 80
_EPT = _CPT * _CHUNK
_EPAD = _EPT * _NW
_NPAD = 10112
_ZR = _NPAD // _NS


def _elu(x):
    return jnp.where(x > 0, x, jnp.exp(jnp.minimum(x, 0.0)) - 1.0)


def _dot(a, b):
    return jnp.dot(a, b, preferred_element_type=jnp.float32)


@functools.lru_cache(maxsize=None)
def _make_seg_sum(dp):
    mesh = plsc.VectorSubcoreMesh(core_axis_name="c", subcore_axis_name="s",
                                  num_cores=_NC, num_subcores=_NS)

    @functools.partial(
        pl.kernel,
        out_type=jax.ShapeDtypeStruct((_NC, _NPAD, dp), jnp.float32),
        mesh=mesh,
        scratch_types=[
            pltpu.VMEM((_CPT, _CHUNK), jnp.int32),
            pltpu.VMEM((_CPT, _CHUNK), jnp.int32),
            [pltpu.VMEM((_CHUNK, dp), jnp.float32) for _ in range(_NB)],
            [pltpu.SemaphoreType.DMA for _ in range(_NB)],
            [pltpu.SemaphoreType.DMA for _ in range(_NB)],
            pltpu.VMEM_SHARED((_NPAD, dp), jnp.float32),
        ],
        compiler_params=pltpu.CompilerParams(use_tc_tiling_on_sc=False),
    )
    def seg_sum(y, ar, zeros, out, sidx, didx, rows, gsem, ssem, acc):
        c = lax.axis_index("c")
        s = lax.axis_index("s")
        wid = s * _NC + c
        pltpu.sync_copy(zeros.at[pl.ds(s * _ZR, _ZR)], acc.at[pl.ds(s * _ZR, _ZR)])
        pltpu.sync_copy(ar.at[0, wid], sidx)
        pltpu.sync_copy(ar.at[1, wid], didx)
        plsc.subcore_barrier()

        for b in range(_NB):
            pltpu.async_copy(y.at[sidx.at[b]], rows[b], gsem[b])

        nj = _CPT // _NB

        def body(j, carry):
            base = j * _NB
            for b in range(_NB):
                i = base + b
                pltpu.make_async_copy(y.at[sidx.at[i]], rows[b], gsem[b]).wait()
                pltpu.async_copy(rows[b], acc.at[didx.at[i]], ssem[b], add=True)

            @pl.when(j + 1 < nj)
            def _():
                for b in range(_NB):
                    i = base + b
                    pltpu.make_async_copy(rows[b], acc.at[didx.at[i]], ssem[b]).wait()
                    pltpu.async_copy(y.at[sidx.at[i + _NB]], rows[b], gsem[b])

            return carry

        lax.fori_loop(0, nj, body, 0)
        for b in range(_NB):
            i = _CPT - _NB + b
            pltpu.make_async_copy(rows[b], acc.at[didx.at[i]], ssem[b]).wait()
        plsc.subcore_barrier()
        pltpu.sync_copy(acc.at[pl.ds(s * _ZR, _ZR)], out.at[c, pl.ds(s * _ZR, _ZR)])

    return seg_sum


def _enc_body(oh, ft, w1a, w1b, b1, w2, b2, w3, b3, wg1, out):
    x = _elu(_dot(oh[...], w1a[...]) + _dot(ft[...], w1b[...]) + b1[...])
    x = _elu(_dot(x, w2[...]) + b2[...])
    x = _elu(_dot(x, w3[...]) + b3[...])
    out[...] = _dot(x, wg1[...])


def _encoder(oh, ft, w1a, w1b, b1, w2, b2, w3, b3, wg1):
    return pl.pallas_call(
        _enc_body,
        out_shape=jax.ShapeDtypeStruct((_N, 64), jnp.float32),
    )(oh, ft, w1a, w1b, b1, w2, b2, w3, b3, wg1)


def _mid_body(p, b, w, out):
    x = _elu(p[0, :_N] + p[1, :_N] + b[...])
    out[...] = _dot(x, w[...])


def _mid(p, b, w):
    dn = w.shape[-1]
    return pl.pallas_call(
        _mid_body,
        out_shape=jax.ShapeDtypeStruct((_N, dn), jnp.float32),
    )(p, b, w)


def _fin_body(p, bg, w1, b1, w2, b2, w3, b3, out):
    x = _elu(p[0, :_N] + p[1, :_N] + bg[...])
    x = _elu(_dot(x, w1[...]) + b1[...])
    x = _elu(_dot(x, w2[...]) + b2[...])
    out[...] = jax.nn.sigmoid(_dot(x, w3[...]) + b3[...])


def _final(p, bg, w1, b1, w2, b2, w3, b3):
    return pl.pallas_call(
        _fin_body,
        out_shape=jax.ShapeDtypeStruct((_N, 1), jnp.float32),
    )(p, bg, w1, b1, w2, b2, w3, b3)


def kernel(one_hot, features, gemme_features, a_res,
           We1, be1, We2, be2, We3, be3,
           Wg1, bg1, Wg2, bg2, Wg3, bg3, Wg4, bg4, Wg5, bg5,
           Wf1, bf1, Wf2, bf2, Wf3, bf3):
    ar = a_res.reshape(2, _NW, _CPT, _CHUNK)

    row = lambda b: b.reshape(1, -1)
    Wg4p = jnp.pad(Wg4, ((0, 0), (0, 8)))
    bg4p = row(jnp.pad(bg4, (0, 8)))
    Wg5p = jnp.pad(Wg5, ((0, 8), (0, 12)))
    bg5p = row(jnp.pad(bg5, (0, 12)))
    Wf1p = jnp.pad(Wf1, ((0, 12), (0, 0)))

    z64 = jnp.zeros((_NPAD, 64), jnp.float32)
    z32 = jnp.zeros((_NPAD, 32), jnp.float32)
    z16 = jnp.zeros((_NPAD, 16), jnp.float32)

    y = _encoder(one_hot, features, We1[:20], We1[20:], row(be1),
                 We2, row(be2), We3, row(be3), Wg1)
    p = _make_seg_sum(64)(y, ar, z64)
    y = _mid(p, row(bg1), Wg2)
    p = _make_seg_sum(32)(y, ar, z32)
    y = _mid(p, row(bg2), Wg3)
    p = _make_seg_sum(16)(y, ar, z16)
    y = _mid(p, row(bg3), Wg4p)
    p = _make_seg_sum(16)(y, ar, z16)
    y = _mid(p, bg4p, Wg5p)
    p = _make_seg_sum(16)(y, ar, z16)
    return _final(p, bg5p, Wf1p, row(bf1), Wf2, row(bf2), Wf3, row(bf3))


# speedup vs baseline: 22.1233x; 1.0807x over previous
import functools

import jax
import jax.numpy as jnp
from jax import lax
from jax.experimental import pallas as pl
from jax.experimental.pallas import tpu as pltpu
from jax.experimental.pallas import tpu_sc as plsc

_N = 10000
_E = 320000
_NC = 2
_NS = 16
_NW = _NC * _NS
_CHUNK = 125
_NB = 8
_CPT = 80
_EPT = _CPT * _CHUNK
_EPAD = _EPT * _NW
_NPAD = 10112
_ZR = _NPAD // _NS


def _elu(x):
    return jnp.where(x > 0, x, jnp.exp(jnp.minimum(x, 0.0)) - 1.0)


def _dot(a, b):
    return jnp.dot(a, b, preferred_element_type=jnp.float32)


@functools.lru_cache(maxsize=None)
def _make_seg_sum(dp):
    mesh = plsc.VectorSubcoreMesh(core_axis_name="c", subcore_axis_name="s",
                                  num_cores=_NC, num_subcores=_NS)

    @functools.partial(
        pl.kernel,
        out_type=jax.ShapeDtypeStruct((_NC, _NPAD, dp), jnp.float32),
        mesh=mesh,
        scratch_types=[
            pltpu.VMEM((_CPT, _CHUNK), jnp.int32),
            pltpu.VMEM((_CPT, _CHUNK), jnp.int32),
            [pltpu.VMEM((_CHUNK, dp), jnp.float32) for _ in range(_NB)],
            [pltpu.SemaphoreType.DMA for _ in range(_NB)],
            [pltpu.SemaphoreType.DMA for _ in range(_NB)],
            pltpu.VMEM_SHARED((_NPAD, dp), jnp.float32),
        ],
        compiler_params=pltpu.CompilerParams(use_tc_tiling_on_sc=False),
    )
    def seg_sum(y, ar, zeros, out, sidx, didx, rows, gsem, ssem, acc):
        c = lax.axis_index("c")
        s = lax.axis_index("s")
        wid = s * _NC + c
        pltpu.sync_copy(zeros.at[pl.ds(s * _ZR, _ZR)], acc.at[pl.ds(s * _ZR, _ZR)])
        pltpu.sync_copy(ar.at[0, wid], sidx)
        pltpu.sync_copy(ar.at[1, wid], didx)
        plsc.subcore_barrier()

        for b in range(_NB):
            pltpu.async_copy(y.at[sidx.at[b]], rows[b], gsem[b])

        nj = _CPT // _NB

        def body(j, carry):
            base = j * _NB
            for b in range(_NB):
                i = base + b
                pltpu.make_async_copy(y.at[sidx.at[i]], rows[b], gsem[b]).wait()
                pltpu.async_copy(rows[b], acc.at[didx.at[i]], ssem[b], add=True)

            @pl.when(j + 1 < nj)
            def _():
                for b in range(_NB):
                    i = base + b
                    pltpu.make_async_copy(rows[b], acc.at[didx.at[i]], ssem[b]).wait()
                    pltpu.async_copy(y.at[sidx.at[i + _NB]], rows[b], gsem[b])

            return carry

        lax.fori_loop(0, nj, body, 0)
        for b in range(_NB):
            i = _CPT - _NB + b
            pltpu.make_async_copy(rows[b], acc.at[didx.at[i]], ssem[b]).wait()
        plsc.subcore_barrier()
        pltpu.sync_copy(acc.at[pl.ds(s * _ZR, _ZR)], out.at[c, pl.ds(s * _ZR, _ZR)])

    return seg_sum


def _enc_body(oh, ft, w1a, w1b, b1, w2, b2, w3, b3, wg1, out):
    x = _elu(_dot(oh[...], w1a[...]) + _dot(ft[...], w1b[...]) + b1[...])
    x = _elu(_dot(x, w2[...]) + b2[...])
    x = _elu(_dot(x, w3[...]) + b3[...])
    out[...] = _dot(x, wg1[...])


def _encoder(oh, ft, w1a, w1b, b1, w2, b2, w3, b3, wg1):
    return pl.pallas_call(
        _enc_body,
        out_shape=jax.ShapeDtypeStruct((_N, 64), jnp.float32),
    )(oh, ft, w1a, w1b, b1, w2, b2, w3, b3, wg1)


def _mid_body(p, b, w, out):
    x = _elu(p[0, :_N] + p[1, :_N] + b[...])
    out[...] = _dot(x, w[...])


def _mid(p, b, w):
    dn = w.shape[-1]
    return pl.pallas_call(
        _mid_body,
        out_shape=jax.ShapeDtypeStruct((_N, dn), jnp.float32),
    )(p, b, w)


def _fin_body(p, bg, w1, b1, w2, b2, w3, b3, out):
    x = _elu(p[0, :_N] + p[1, :_N] + bg[...])
    x = _elu(_dot(x, w1[...]) + b1[...])
    x = _elu(_dot(x, w2[...]) + b2[...])
    out[...] = jax.nn.sigmoid(_dot(x, w3[...]) + b3[...])


def _final(p, bg, w1, b1, w2, b2, w3, b3):
    return pl.pallas_call(
        _fin_body,
        out_shape=jax.ShapeDtypeStruct((_N, 1), jnp.float32),
    )(p, bg, w1, b1, w2, b2, w3, b3)


def kernel(one_hot, features, gemme_features, a_res,
           We1, be1, We2, be2, We3, be3,
           Wg1, bg1, Wg2, bg2, Wg3, bg3, Wg4, bg4, Wg5, bg5,
           Wf1, bf1, Wf2, bf2, Wf3, bf3):
    ar = a_res.reshape(2, _NW, _CPT, _CHUNK)

    row = lambda b: b.reshape(1, -1)
    Wg4p = jnp.pad(Wg4, ((0, 0), (0, 8)))
    bg4p = row(jnp.pad(bg4, (0, 8)))
    Wg5p = jnp.pad(Wg5, ((0, 8), (0, 12)))
    bg5p = row(jnp.pad(bg5, (0, 12)))
    Wf1p = jnp.pad(Wf1, ((0, 12), (0, 0)))

    z64 = jnp.zeros((_NPAD, 64), jnp.float32)
    z32 = jnp.zeros((_NPAD, 32), jnp.float32)
    z16 = jnp.zeros((_NPAD, 16), jnp.float32)

    y = _encoder(one_hot, features, We1[:20], We1[20:], row(be1),
                 We2, row(be2), We3, row(be3), Wg1)
    p = _make_seg_sum(64)(y, ar, z64)
    y = _mid(p, row(bg1), Wg2)
    p = _make_seg_sum(32)(y, ar, z32)
    y = _mid(p, row(bg2), Wg3)
    p = _make_seg_sum(16)(y, ar, z16)
    y = _mid(p, row(bg3), Wg4p)
    p = _make_seg_sum(16)(y, ar, z16)
    y = _mid(p, bg4p, Wg5p)
    p = _make_seg_sum(16)(y, ar, z16)
    return _final(p, bg5p, Wf1p, row(bf1), Wf2, row(bf2), Wf3, row(bf3))


# speedup vs baseline: 22.1419x; 1.0008x over previous
import functools

import jax
import jax.numpy as jnp
from jax import lax
from jax.experimental import pallas as pl
from jax.experimental.pallas import tpu as pltpu
from jax.experimental.pallas import tpu_sc as plsc

_N = 10000
_E = 320000
_NC = 2
_NS = 16
_NW = _NC * _NS
_CHUNK = 125
_NB = 8
_CPT = 80
_EPT = _CPT * _CHUNK
_EPAD = _EPT * _NW
_NPAD = 10112
_ZR = _NPAD // _NS


def _elu(x):
    return jnp.where(x > 0, x, jnp.exp(jnp.minimum(x, 0.0)) - 1.0)


def _dot(a, b):
    return jnp.dot(a, b, preferred_element_type=jnp.float32)


@functools.lru_cache(maxsize=None)
def _make_seg_sum(dp):
    mesh = plsc.VectorSubcoreMesh(core_axis_name="c", subcore_axis_name="s",
                                  num_cores=_NC, num_subcores=_NS)
    nb = _NB

    @functools.partial(
        pl.kernel,
        out_type=jax.ShapeDtypeStruct((_NC, _NPAD, dp), jnp.float32),
        mesh=mesh,
        scratch_types=[
            pltpu.VMEM((_CPT, _CHUNK), jnp.int32),
            pltpu.VMEM((_CPT, _CHUNK), jnp.int32),
            [pltpu.VMEM((_CHUNK, dp), jnp.float32) for _ in range(nb)],
            [pltpu.SemaphoreType.DMA for _ in range(nb)],
            [pltpu.SemaphoreType.DMA for _ in range(nb)],
            pltpu.VMEM_SHARED((_NPAD, dp), jnp.float32),
        ],
        compiler_params=pltpu.CompilerParams(use_tc_tiling_on_sc=False),
    )
    def seg_sum(y, ar, zeros, out, sidx, didx, rows, gsem, ssem, acc):
        c = lax.axis_index("c")
        s = lax.axis_index("s")
        wid = s * _NC + c
        pltpu.sync_copy(zeros.at[pl.ds(s * _ZR, _ZR)], acc.at[pl.ds(s * _ZR, _ZR)])
        pltpu.sync_copy(ar.at[0, wid], sidx)
        pltpu.sync_copy(ar.at[1, wid], didx)
        plsc.subcore_barrier()

        for b in range(nb):
            pltpu.async_copy(y.at[sidx.at[b]], rows[b], gsem[b])

        nj = _CPT // nb

        def body(j, carry):
            base = j * nb
            for b in range(nb):
                i = base + b
                pltpu.make_async_copy(y.at[sidx.at[i]], rows[b], gsem[b]).wait()
                pltpu.async_copy(rows[b], acc.at[didx.at[i]], ssem[b], add=True)

            @pl.when(j + 1 < nj)
            def _():
                for b in range(nb):
                    i = base + b
                    pltpu.make_async_copy(rows[b], acc.at[didx.at[i]], ssem[b]).wait()
                    pltpu.async_copy(y.at[sidx.at[i + nb]], rows[b], gsem[b])

            return carry

        lax.fori_loop(0, nj, body, 0)
        for b in range(nb):
            i = _CPT - nb + b
            pltpu.make_async_copy(rows[b], acc.at[didx.at[i]], ssem[b]).wait()
        plsc.subcore_barrier()
        pltpu.sync_copy(acc.at[pl.ds(s * _ZR, _ZR)], out.at[c, pl.ds(s * _ZR, _ZR)])

    return seg_sum


def _enc_body(oh, ft, w1a, w1b, b1, w2, b2, w3, b3, wg1, out):
    x = _elu(_dot(oh[...], w1a[...]) + _dot(ft[...], w1b[...]) + b1[...])
    x = _elu(_dot(x, w2[...]) + b2[...])
    x = _elu(_dot(x, w3[...]) + b3[...])
    out[...] = _dot(x, wg1[...])


def _encoder(oh, ft, w1a, w1b, b1, w2, b2, w3, b3, wg1):
    return pl.pallas_call(
        _enc_body,
        out_shape=jax.ShapeDtypeStruct((_N, 64), jnp.float32),
    )(oh, ft, w1a, w1b, b1, w2, b2, w3, b3, wg1)


def _mid_body(p, b, w, out):
    x = _elu(p[0, :_N] + p[1, :_N] + b[...])
    out[...] = _dot(x, w[...])


def _mid(p, b, w):
    dn = w.shape[-1]
    return pl.pallas_call(
        _mid_body,
        out_shape=jax.ShapeDtypeStruct((_N, dn), jnp.float32),
    )(p, b, w)


def _fin_body(p, bg, w1, b1, w2, b2, w3, b3, out):
    x = _elu(p[0, :_N] + p[1, :_N] + bg[...])
    x = _elu(_dot(x, w1[...]) + b1[...])
    x = _elu(_dot(x, w2[...]) + b2[...])
    out[...] = jax.nn.sigmoid(_dot(x, w3[...]) + b3[...])


def _final(p, bg, w1, b1, w2, b2, w3, b3):
    return pl.pallas_call(
        _fin_body,
        out_shape=jax.ShapeDtypeStruct((_N, 1), jnp.float32),
    )(p, bg, w1, b1, w2, b2, w3, b3)


def kernel(one_hot, features, gemme_features, a_res,
           We1, be1, We2, be2, We3, be3,
           Wg1, bg1, Wg2, bg2, Wg3, bg3, Wg4, bg4, Wg5, bg5,
           Wf1, bf1, Wf2, bf2, Wf3, bf3):
    ar = a_res.reshape(2, _NW, _CPT, _CHUNK)

    row = lambda b: b.reshape(1, -1)
    Wg4p = jnp.pad(Wg4, ((0, 0), (0, 8)))
    bg4p = row(jnp.pad(bg4, (0, 8)))
    Wg5p = jnp.pad(Wg5, ((0, 8), (0, 12)))
    bg5p = row(jnp.pad(bg5, (0, 12)))
    Wf1p = jnp.pad(Wf1, ((0, 12), (0, 0)))

    z64 = jnp.zeros((_NPAD, 64), jnp.float32)
    z32 = jnp.zeros((_NPAD, 32), jnp.float32)
    z16 = jnp.zeros((_NPAD, 16), jnp.float32)

    y = _encoder(one_hot, features, We1[:20], We1[20:], row(be1),
                 We2, row(be2), We3, row(be3), Wg1)
    p = _make_seg_sum(64)(y, ar, z64)
    y = _mid(p, row(bg1), Wg2)
    p = _make_seg_sum(32)(y, ar, z32)
    y = _mid(p, row(bg2), Wg3)
    p = _make_seg_sum(16)(y, ar, z16)
    y = _mid(p, row(bg3), Wg4p)
    p = _make_seg_sum(16)(y, ar, z16)
    y = _mid(p, bg4p, Wg5p)
    p = _make_seg_sum(16)(y, ar, z16)
    return _final(p, bg5p, Wf1p, row(bf1), Wf2, row(bf2), Wf3, row(bf3))


# speedup vs baseline: 22.5511x; 1.0185x over previous
import functools

import jax
import jax.numpy as jnp
from jax import lax
from jax.experimental import pallas as pl
from jax.experimental.pallas import tpu as pltpu
from jax.experimental.pallas import tpu_sc as plsc

_N = 10000
_E = 320000
_NC = 2
_NS = 16
_NW = _NC * _NS
_CHUNK = 250
_NB = 8
_CPT = 40
_EPT = _CPT * _CHUNK
_EPAD = _EPT * _NW
_NPAD = 10112
_ZR = _NPAD // _NS


def _elu(x):
    return jnp.where(x > 0, x, jnp.exp(jnp.minimum(x, 0.0)) - 1.0)


def _dot(a, b):
    return jnp.dot(a, b, preferred_element_type=jnp.float32)


@functools.lru_cache(maxsize=None)
def _make_seg_sum(dp):
    mesh = plsc.VectorSubcoreMesh(core_axis_name="c", subcore_axis_name="s",
                                  num_cores=_NC, num_subcores=_NS)
    nb = _NB if dp < 64 else 4

    @functools.partial(
        pl.kernel,
        out_type=jax.ShapeDtypeStruct((_NC, _NPAD, dp), jnp.float32),
        mesh=mesh,
        scratch_types=[
            pltpu.VMEM((_CPT, _CHUNK), jnp.int32),
            pltpu.VMEM((_CPT, _CHUNK), jnp.int32),
            [pltpu.VMEM((_CHUNK, dp), jnp.float32) for _ in range(nb)],
            [pltpu.SemaphoreType.DMA for _ in range(nb)],
            [pltpu.SemaphoreType.DMA for _ in range(nb)],
            pltpu.VMEM_SHARED((_NPAD, dp), jnp.float32),
        ],
        compiler_params=pltpu.CompilerParams(use_tc_tiling_on_sc=False),
    )
    def seg_sum(y, ar, zeros, out, sidx, didx, rows, gsem, ssem, acc):
        c = lax.axis_index("c")
        s = lax.axis_index("s")
        wid = s * _NC + c
        pltpu.sync_copy(zeros.at[pl.ds(s * _ZR, _ZR)], acc.at[pl.ds(s * _ZR, _ZR)])
        pltpu.sync_copy(ar.at[0, wid], sidx)
        pltpu.sync_copy(ar.at[1, wid], didx)
        plsc.subcore_barrier()

        for b in range(nb):
            pltpu.async_copy(y.at[sidx.at[b]], rows[b], gsem[b])

        nj = _CPT // nb

        def body(j, carry):
            base = j * nb
            for b in range(nb):
                i = base + b
                pltpu.make_async_copy(y.at[sidx.at[i]], rows[b], gsem[b]).wait()
                pltpu.async_copy(rows[b], acc.at[didx.at[i]], ssem[b], add=True)

            @pl.when(j + 1 < nj)
            def _():
                for b in range(nb):
                    i = base + b
                    pltpu.make_async_copy(rows[b], acc.at[didx.at[i]], ssem[b]).wait()
                    pltpu.async_copy(y.at[sidx.at[i + nb]], rows[b], gsem[b])

            return carry

        lax.fori_loop(0, nj, body, 0)
        for b in range(nb):
            i = _CPT - nb + b
            pltpu.make_async_copy(rows[b], acc.at[didx.at[i]], ssem[b]).wait()
        plsc.subcore_barrier()
        pltpu.sync_copy(acc.at[pl.ds(s * _ZR, _ZR)], out.at[c, pl.ds(s * _ZR, _ZR)])

    return seg_sum


def _enc_body(oh, ft, w1a, w1b, b1, w2, b2, w3, b3, wg1, out):
    x = _elu(_dot(oh[...], w1a[...]) + _dot(ft[...], w1b[...]) + b1[...])
    x = _elu(_dot(x, w2[...]) + b2[...])
    x = _elu(_dot(x, w3[...]) + b3[...])
    out[...] = _dot(x, wg1[...])


def _encoder(oh, ft, w1a, w1b, b1, w2, b2, w3, b3, wg1):
    return pl.pallas_call(
        _enc_body,
        out_shape=jax.ShapeDtypeStruct((_N, 64), jnp.float32),
    )(oh, ft, w1a, w1b, b1, w2, b2, w3, b3, wg1)


def _mid_body(p, b, w, out):
    x = _elu(p[0, :_N] + p[1, :_N] + b[...])
    out[...] = _dot(x, w[...])


def _mid(p, b, w):
    dn = w.shape[-1]
    return pl.pallas_call(
        _mid_body,
        out_shape=jax.ShapeDtypeStruct((_N, dn), jnp.float32),
    )(p, b, w)


def _fin_body(p, bg, w1, b1, w2, b2, w3, b3, out):
    x = _elu(p[0, :_N] + p[1, :_N] + bg[...])
    x = _elu(_dot(x, w1[...]) + b1[...])
    x = _elu(_dot(x, w2[...]) + b2[...])
    out[...] = jax.nn.sigmoid(_dot(x, w3[...]) + b3[...])


def _final(p, bg, w1, b1, w2, b2, w3, b3):
    return pl.pallas_call(
        _fin_body,
        out_shape=jax.ShapeDtypeStruct((_N, 1), jnp.float32),
    )(p, bg, w1, b1, w2, b2, w3, b3)


def kernel(one_hot, features, gemme_features, a_res,
           We1, be1, We2, be2, We3, be3,
           Wg1, bg1, Wg2, bg2, Wg3, bg3, Wg4, bg4, Wg5, bg5,
           Wf1, bf1, Wf2, bf2, Wf3, bf3):
    ar = a_res.reshape(2, _NW, _CPT, _CHUNK)

    row = lambda b: b.reshape(1, -1)
    Wg4p = jnp.pad(Wg4, ((0, 0), (0, 8)))
    bg4p = row(jnp.pad(bg4, (0, 8)))
    Wg5p = jnp.pad(Wg5, ((0, 8), (0, 12)))
    bg5p = row(jnp.pad(bg5, (0, 12)))
    Wf1p = jnp.pad(Wf1, ((0, 12), (0, 0)))

    z64 = jnp.zeros((_NPAD, 64), jnp.float32)
    z32 = jnp.zeros((_NPAD, 32), jnp.float32)
    z16 = jnp.zeros((_NPAD, 16), jnp.float32)

    y = _encoder(one_hot, features, We1[:20], We1[20:], row(be1),
                 We2, row(be2), We3, row(be3), Wg1)
    p = _make_seg_sum(64)(y, ar, z64)
    y = _mid(p, row(bg1), Wg2)
    p = _make_seg_sum(32)(y, ar, z32)
    y = _mid(p, row(bg2), Wg3)
    p = _make_seg_sum(16)(y, ar, z16)
    y = _mid(p, row(bg3), Wg4p)
    p = _make_seg_sum(16)(y, ar, z16)
    y = _mid(p, bg4p, Wg5p)
    p = _make_seg_sum(16)(y, ar, z16)
    return _final(p, bg5p, Wf1p, row(bf1), Wf2, row(bf2), Wf3, row(bf3))
